# baseline (speedup 1.0000x reference)
import functools

import jax
import jax.numpy as jnp
from jax import lax
from jax.experimental import pallas as pl
from jax.experimental.pallas import tpu as pltpu
from jax.experimental.pallas import tpu_sc as plsc

_NC = 2
_NS = 16
_NW = _NC * _NS


def _gather4(uidx, iidx, t_umf, t_imf, t_umlp, t_imlp):
    B = uidx.shape[0]
    D = t_umf.shape[1]
    bw = B // _NW

    mesh = plsc.VectorSubcoreMesh(
        core_axis_name="c", subcore_axis_name="s",
        num_cores=_NC, num_subcores=_NS)

    @functools.partial(
        pl.kernel,
        mesh=mesh,
        out_type=[jax.ShapeDtypeStruct((B, D), jnp.float32)] * 4,
        scratch_types=[
            pltpu.VMEM((bw,), jnp.int32),
            pltpu.VMEM((bw,), jnp.int32),
            pltpu.VMEM((bw, D), jnp.float32),
            pltpu.VMEM((bw, D), jnp.float32),
            pltpu.SemaphoreType.DMA,
            pltpu.SemaphoreType.DMA,
        ],
        compiler_params=pltpu.CompilerParams(use_tc_tiling_on_sc=False),
    )
    def k(uidx_hbm, iidx_hbm, umf_hbm, imf_hbm, umlp_hbm, imlp_hbm,
          out_umf, out_imf, out_umlp, out_imlp,
          uidx_v, iidx_v, buf0, buf1, sem0, sem1):
        wid = lax.axis_index("s") * _NC + lax.axis_index("c")
        base = wid * bw
        pltpu.sync_copy(uidx_hbm.at[pl.ds(base, bw)], uidx_v)
        pltpu.sync_copy(iidx_hbm.at[pl.ds(base, bw)], iidx_v)
        cp0 = pltpu.async_copy(umf_hbm.at[uidx_v], buf0, sem0)
        cp1 = pltpu.async_copy(imf_hbm.at[iidx_v], buf1, sem1)
        cp0.wait()
        pltpu.sync_copy(buf0, out_umf.at[pl.ds(base, bw)])
        cp0 = pltpu.async_copy(umlp_hbm.at[uidx_v], buf0, sem0)
        cp1.wait()
        pltpu.sync_copy(buf1, out_imf.at[pl.ds(base, bw)])
        cp1 = pltpu.async_copy(imlp_hbm.at[iidx_v], buf1, sem1)
        cp0.wait()
        pltpu.sync_copy(buf0, out_umlp.at[pl.ds(base, bw)])
        cp1.wait()
        pltpu.sync_copy(buf1, out_imlp.at[pl.ds(base, bw)])

    return k(uidx, iidx, t_umf, t_imf, t_umlp, t_imlp)


def _tower_body(umlp, imlp, umf, imf, w1a, w1b, b1, w2, b2, w3, b3,
                womlp, womf, bo, out):
    h = (jnp.dot(umlp[...], w1a[...], preferred_element_type=jnp.float32)
         + jnp.dot(imlp[...], w1b[...], preferred_element_type=jnp.float32)
         + b1[...])
    h = jnp.maximum(h, 0.0)
    h = jnp.maximum(
        jnp.dot(h, w2[...], preferred_element_type=jnp.float32) + b2[...], 0.0)
    h = jnp.maximum(
        jnp.dot(h, w3[...], preferred_element_type=jnp.float32) + b3[...], 0.0)
    mf = umf[...] * imf[...]
    logit = (jnp.sum(h * womlp[...], axis=-1, keepdims=True)
             + jnp.sum(mf * womf[...], axis=-1, keepdims=True)
             + bo[...])
    out[...] = jax.nn.sigmoid(logit)


def _tower(umlp, imlp, umf, imf, W1, b1, W2, b2, W3, b3, W_out, b_out):
    B, D = umf.shape
    grid = 8
    bm = B // grid
    w1a, w1b = W1[:D], W1[D:]
    womlp = W_out[:16, 0].reshape(1, 16)
    womf = W_out[16:, 0].reshape(1, D)

    full = lambda i: (0, 0)
    row = lambda i: (i, 0)
    out = pl.pallas_call(
        _tower_body,
        grid=(grid,),
        in_specs=[
            pl.BlockSpec((bm, D), row),
            pl.BlockSpec((bm, D), row),
            pl.BlockSpec((bm, D), row),
            pl.BlockSpec((bm, D), row),
            pl.BlockSpec((D, 64), full),
            pl.BlockSpec((D, 64), full),
            pl.BlockSpec((1, 64), full),
            pl.BlockSpec((64, 32), full),
            pl.BlockSpec((1, 32), full),
            pl.BlockSpec((32, 16), full),
            pl.BlockSpec((1, 16), full),
            pl.BlockSpec((1, 16), full),
            pl.BlockSpec((1, D), full),
            pl.BlockSpec((1, 1), full),
        ],
        out_specs=pl.BlockSpec((bm, 1), row),
        out_shape=jax.ShapeDtypeStruct((B, 1), jnp.float32),
        compiler_params=pltpu.CompilerParams(
            dimension_semantics=("arbitrary",)),
    )(umlp, imlp, umf, imf, w1a, w1b, b1.reshape(1, -1), W2,
      b2.reshape(1, -1), W3, b3.reshape(1, -1), womlp, womf,
      b_out.reshape(1, 1))
    return out[:, 0]


def kernel(user_indices, item_indices, emb_user_mf, emb_item_mf,
           emb_user_mlp, emb_item_mlp, W1, b1, W2, b2, W3, b3, W_out, b_out):
    ui = user_indices.astype(jnp.int32)
    ii = item_indices.astype(jnp.int32)
    umf, imf, umlp, imlp = _gather4(
        ui, ii, emb_user_mf, emb_item_mf, emb_user_mlp, emb_item_mlp)
    return _tower(umlp, imlp, umf, imf, W1, b1, W2, b2, W3, b3, W_out, b_out)


# speedup vs baseline: 1.5711x; 1.5711x over previous
import functools

import jax
import jax.numpy as jnp
from jax import lax
from jax.experimental import pallas as pl
from jax.experimental.pallas import tpu as pltpu
from jax.experimental.pallas import tpu_sc as plsc

_NC = 2
_NS = 16
_NW = _NC * _NS
_CHUNK = 256


_SPLIT = 500224


def _pair_body(xa0, xb0, xa1, xb1, xa2, xb2, xa3, xb3, o0, o1, o2, o3):
    for xa, xb, o in ((xa0, xb0, o0), (xa1, xb1, o1),
                      (xa2, xb2, o2), (xa3, xb3, o3)):
        o[:, :64] = xa[...].T
        o[:, 64:] = xb[...].T


def _pair4(t0, t1, t2, t3):
    D, N = t0.shape
    bw = 512
    grid = _SPLIT // bw
    spec_a = pl.BlockSpec((D, bw), lambda j: (0, j))
    spec_b = pl.BlockSpec((D, bw), lambda j: (0, j + _SPLIT // 512))
    spec_out = pl.BlockSpec((bw, 2 * D), lambda j: (j, 0))
    ins = []
    for t in (t0, t1, t2, t3):
        ins.extend([t, t])
    return pl.pallas_call(
        _pair_body,
        grid=(grid,),
        in_specs=[spec_a, spec_b] * 4,
        out_specs=[spec_out] * 4,
        out_shape=[jax.ShapeDtypeStruct((_SPLIT, 2 * D), jnp.float32)] * 4,
        compiler_params=pltpu.CompilerParams(
            dimension_semantics=("arbitrary",)),
    )(*ins)


def _gather4(puidx, piidx, p_umf, p_imf, p_umlp, p_imlp):
    B = puidx.shape[0]
    D = p_umf.shape[1]
    bw = B // _NW
    nchunk = bw // _CHUNK

    mesh = plsc.VectorSubcoreMesh(
        core_axis_name="c", subcore_axis_name="s",
        num_cores=_NC, num_subcores=_NS)

    @functools.partial(
        pl.kernel,
        mesh=mesh,
        out_type=[jax.ShapeDtypeStruct((B, D), jnp.float32)] * 4,
        scratch_types=[
            pltpu.VMEM((bw,), jnp.int32),
            pltpu.VMEM((bw,), jnp.int32),
            pltpu.VMEM((_CHUNK, D), jnp.float32),
            pltpu.VMEM((_CHUNK, D), jnp.float32),
            pltpu.SemaphoreType.DMA,
            pltpu.SemaphoreType.DMA,
        ],
    )
    def k(uidx_hbm, iidx_hbm, umf_hbm, imf_hbm, umlp_hbm, imlp_hbm,
          out_umf, out_imf, out_umlp, out_imlp,
          uidx_v, iidx_v, buf0, buf1, sem0, sem1):
        wid = lax.axis_index("s") * _NC + lax.axis_index("c")
        base = wid * bw
        pltpu.sync_copy(uidx_hbm.at[pl.ds(base, bw)], uidx_v)
        pltpu.sync_copy(iidx_hbm.at[pl.ds(base, bw)], iidx_v)

        def chunk_body(c, _):
            cb = c * _CHUNK
            iu = uidx_v.at[pl.ds(cb, _CHUNK)]
            ii = iidx_v.at[pl.ds(cb, _CHUNK)]
            od = pl.ds(base + cb, _CHUNK)
            cp0 = pltpu.async_copy(umf_hbm.at[iu], buf0, sem0)
            cp1 = pltpu.async_copy(imf_hbm.at[ii], buf1, sem1)
            cp0.wait()
            pltpu.sync_copy(buf0, out_umf.at[od])
            cp0 = pltpu.async_copy(umlp_hbm.at[iu], buf0, sem0)
            cp1.wait()
            pltpu.sync_copy(buf1, out_imf.at[od])
            cp1 = pltpu.async_copy(imlp_hbm.at[ii], buf1, sem1)
            cp0.wait()
            pltpu.sync_copy(buf0, out_umlp.at[od])
            cp1.wait()
            pltpu.sync_copy(buf1, out_imlp.at[od])
            return _

        lax.fori_loop(0, nchunk, chunk_body, None)

    return k(puidx, piidx, p_umf, p_imf, p_umlp, p_imlp)


def _tower_body(gumlp, gimlp, gumf, gimf, upar, ipar, w1a, w1b, b1, w2, b2,
                w3, b3, womlp, womf, bo, out):
    usel = upar[...] == 0
    isel = ipar[...] == 0
    umlp = jnp.where(usel, gumlp[:, :64], gumlp[:, 64:])
    imlp = jnp.where(isel, gimlp[:, :64], gimlp[:, 64:])
    umf = jnp.where(usel, gumf[:, :64], gumf[:, 64:])
    imf = jnp.where(isel, gimf[:, :64], gimf[:, 64:])
    h = (jnp.dot(umlp, w1a[...], preferred_element_type=jnp.float32)
         + jnp.dot(imlp, w1b[...], preferred_element_type=jnp.float32)
         + b1[...])
    h = jnp.maximum(h, 0.0)
    h = jnp.maximum(
        jnp.dot(h, w2[...], preferred_element_type=jnp.float32) + b2[...], 0.0)
    h = jnp.maximum(
        jnp.dot(h, w3[...], preferred_element_type=jnp.float32) + b3[...], 0.0)
    mf = umf * imf
    logit = (jnp.sum(h * womlp[...], axis=-1, keepdims=True)
             + jnp.sum(mf * womf[...], axis=-1, keepdims=True)
             + bo[...])
    out[...] = jax.nn.sigmoid(logit)


def _tower(gumlp, gimlp, gumf, gimf, ui, ii,
           W1, b1, W2, b2, W3, b3, W_out, b_out):
    B = gumf.shape[0]
    D = 64
    grid = 8
    bm = B // grid
    w1a, w1b = W1[:D], W1[D:]
    womlp = W_out[:16, 0].reshape(1, 16)
    womf = W_out[16:, 0].reshape(1, D)

    full = lambda i: (0, 0)
    row = lambda i: (i, 0)
    out = pl.pallas_call(
        _tower_body,
        grid=(grid,),
        in_specs=[
            pl.BlockSpec((bm, 2 * D), row),
            pl.BlockSpec((bm, 2 * D), row),
            pl.BlockSpec((bm, 2 * D), row),
            pl.BlockSpec((bm, 2 * D), row),
            pl.BlockSpec((bm, 1), row),
            pl.BlockSpec((bm, 1), row),
            pl.BlockSpec((D, 64), full),
            pl.BlockSpec((D, 64), full),
            pl.BlockSpec((1, 64), full),
            pl.BlockSpec((64, 32), full),
            pl.BlockSpec((1, 32), full),
            pl.BlockSpec((32, 16), full),
            pl.BlockSpec((1, 16), full),
            pl.BlockSpec((1, 16), full),
            pl.BlockSpec((1, D), full),
            pl.BlockSpec((1, 1), full),
        ],
        out_specs=pl.BlockSpec((bm, 1), row),
        out_shape=jax.ShapeDtypeStruct((B, 1), jnp.float32),
        compiler_params=pltpu.CompilerParams(
            dimension_semantics=("arbitrary",)),
    )(gumlp, gimlp, gumf, gimf, ui, ii, w1a, w1b, b1.reshape(1, -1), W2,
      b2.reshape(1, -1), W3, b3.reshape(1, -1), womlp, womf,
      b_out.reshape(1, 1))
    return out[:, 0]


def kernel(user_indices, item_indices, emb_user_mf, emb_item_mf,
           emb_user_mlp, emb_item_mlp, W1, b1, W2, b2, W3, b3, W_out, b_out):
    ui = user_indices.astype(jnp.int32)
    ii = item_indices.astype(jnp.int32)
    p_umf, p_imf, p_umlp, p_imlp = _pair4(
        emb_user_mf.T, emb_item_mf.T, emb_user_mlp.T, emb_item_mlp.T)
    uh = (ui >= _SPLIT).astype(jnp.int32)
    ih = (ii >= _SPLIT).astype(jnp.int32)
    gumf, gimf, gumlp, gimlp = _gather4(
        ui - uh * _SPLIT, ii - ih * _SPLIT, p_umf, p_imf, p_umlp, p_imlp)
    return _tower(gumlp, gimlp, gumf, gimf,
                  uh.reshape(-1, 1), ih.reshape(-1, 1),
                  W1, b1, W2, b2, W3, b3, W_out, b_out)
